# baseline (speedup 1.0000x reference)
import functools

import jax
import jax.numpy as jnp
from jax import lax
from jax.experimental import pallas as pl
from jax.experimental.pallas import tpu as pltpu
from jax.experimental.pallas import tpu_sc as plsc

NW = 32
CH = 32
L = 20
D = 32
IDX_MINOR = 128
CHL = CH * L
NG = CHL // IDX_MINOR


def _renorm_body(a_ref, oa_ref):
    gi = lax.broadcasted_iota(jnp.int32, (256, 256), 0) >> 5
    gj = lax.broadcasted_iota(jnp.int32, (256, 256), 1) >> 5
    g = jnp.where(gi == gj, 1.0, 0.0).astype(jnp.float32)

    def renorm_pack(x):
        n2 = jnp.dot(x * x, g, preferred_element_type=jnp.float32)
        n = jnp.sqrt(n2)
        scale = jnp.where(n > 10.0, 10.0 / (n + 1e-7), 1.0)
        y = x * scale
        u = lax.bitcast_convert_type(y, jnp.int32)
        r = (u + 0x7FFF + ((u >> 16) & 1)) >> 16
        r = r & 0xFFFF
        lo = jnp.concatenate([r[:, 32 * t:32 * t + 16] for t in range(8)],
                             axis=1)
        hi = jnp.concatenate([r[:, 32 * t + 16:32 * t + 32]
                              for t in range(8)], axis=1)
        return lo | (hi << 16)

    oa_ref[...] = renorm_pack(a_ref[...])


def _renorm2(w_a, w_b):
    v8 = w_a.shape[0]
    bs = 1600
    spec = pl.BlockSpec((bs, 256), lambda i: (i, 0))
    ospec = pl.BlockSpec((bs, 128), lambda i: (i, 0))
    call = pl.pallas_call(
        _renorm_body,
        grid=(v8 // bs,),
        in_specs=[spec],
        out_specs=ospec,
        out_shape=jax.ShapeDtypeStruct((v8, 128), jnp.int32),
    )
    return call(w_a), call(w_b)


MAXSEG = 50


def _windows(total):
    out = []
    off = 0
    while total - off > 128:
        out.append((off, 128))
        off += 128
    out.append((off, total - off))
    return out


def _gather_mean(table_in, table_out, mk_idx, mv_idx, q_idx, negs_idx,
                 resp_idx, s_in, s_out):
    mesh = plsc.VectorSubcoreMesh(core_axis_name="c", subcore_axis_name="s")

    @functools.partial(
        pl.kernel,
        mesh=mesh,
        compiler_params=pltpu.CompilerParams(use_tc_tiling_on_sc=False),
        out_type=[
            jax.ShapeDtypeStruct((s_in, D), jnp.float32),
            jax.ShapeDtypeStruct((s_out, D), jnp.float32),
        ],
        scratch_types=[
            pltpu.VMEM((MAXSEG * L,), jnp.int32),
            pltpu.VMEM((MAXSEG * L,), jnp.int32),
            pltpu.VMEM((MAXSEG * L,), jnp.int32),
            pltpu.VMEM((MAXSEG * L, D // 2), jnp.int32),
            pltpu.VMEM((MAXSEG * L, D // 2), jnp.int32),
            pltpu.VMEM((MAXSEG * L, D // 2), jnp.int32),
            pltpu.VMEM((MAXSEG, D), jnp.float32),
            pltpu.VMEM((20, D), jnp.float32),
            pltpu.VMEM((32, D), jnp.float32),
            pltpu.SemaphoreType.DMA,
            pltpu.SemaphoreType.DMA,
            pltpu.SemaphoreType.DMA,
        ],
    )
    def k(tin, tout, mki, mvi, qi, ngi, rsi, pooled_in, pooled_out,
          idx0, idx1, idx2, rows0, rows1, rows2, out50, out20, out32,
          sem0, sem1, sem2):
        wid = lax.axis_index("s") * 2 + lax.axis_index("c")
        bufs = ((idx0, rows0, sem0), (idx1, rows1, sem1),
                (idx2, rows2, sem2))

        def gather_descs(table, p, nidx):
            idxv, rowsv, sem = bufs[p]
            return [
                (
                    table.at[idxv.at[pl.ds(off, sz)]],
                    rowsv.at[pl.ds(off, sz), :],
                    sem,
                )
                for off, sz in _windows(nidx)
            ]

        def prefetch(table, idx_row, p, nidx):
            pltpu.sync_copy(idx_row, bufs[p][0].at[pl.ds(0, nidx)])
            for src, dst, sem in gather_descs(table, p, nidx):
                pltpu.async_copy(src, dst, sem)

        def reduce_store(rowsv, outv, segs):
            def halves(r):
                w = rowsv[r, pl.ds(0, D // 2)]
                a = lax.bitcast_convert_type(w << 16, jnp.float32)
                b2 = lax.bitcast_convert_type(w & jnp.int32(-65536),
                                              jnp.float32)
                return a, b2

            def seg_body(s, carry):
                base = s * L
                acc0, acc1 = halves(base)
                for l in range(1, L):
                    a, bb = halves(base + l)
                    acc0 = acc0 + a
                    acc1 = acc1 + bb
                outv[s, pl.ds(0, 16)] = acc0 * (1.0 / L)
                outv[s, pl.ds(16, 16)] = acc1 * (1.0 / L)
                return carry

            lax.fori_loop(0, segs, seg_body, 0)

        def process(table, pooled, outv, row0, p, segs):
            for src, dst, sem in gather_descs(table, p, segs * L):
                pltpu.make_async_copy(src, dst, sem).wait()
            reduce_store(bufs[p][1], outv, segs)
            pltpu.sync_copy(outv, pooled.at[pl.ds(row0, segs), :])

        def do_batch_stream(table, idx2d, pooled, outv, row0, segs):
            b0 = wid * 32

            def pre(c, p):
                prefetch(table, idx2d.at[b0 + c], p, segs * L)

            def proc(c, p):
                process(table, pooled, outv, row0 + (b0 + c) * segs, p,
                        segs)

            pre(0, 0)
            pre(1, 1)
            pre(2, 2)

            def ring_body(k3, carry):
                c = 3 * k3
                proc(c, 0)
                pre(c + 3, 0)
                proc(c + 1, 1)
                pre(c + 4, 1)
                proc(c + 2, 2)
                pre(c + 5, 2)
                return carry

            lax.fori_loop(0, 9, ring_body, 0)
            proc(27, 0)
            pre(30, 0)
            proc(28, 1)
            pre(31, 1)
            proc(29, 2)
            proc(30, 0)
            proc(31, 1)

        def do_flat_stream(table, idx_flat, pooled, row0):
            segs = 32
            b0 = wid * segs
            prefetch(table, idx_flat.at[pl.ds(b0 * L, segs * L)], 0,
                     segs * L)
            process(table, pooled, out32, row0 + b0, 0, segs)

        b = mki.shape[0]
        m = mki.shape[1] // L
        n = ngi.shape[1] // L
        do_batch_stream(tin, mki, pooled_in, out50, 0, m)
        do_batch_stream(tin, mvi, pooled_in, out50, b * m, m)
        do_flat_stream(tin, qi, pooled_in, 2 * b * m)
        do_batch_stream(tout, ngi, pooled_out, out20, 0, n)
        do_flat_stream(tout, rsi, pooled_out, b * n)

    return k(table_in, table_out, mk_idx, mv_idx, q_idx, negs_idx, resp_idx)


def _dense_body(mk_ref, mv_ref, q_ref, negs_ref, resp_ref, wl_ref,
                xe_ref, ne_ref):
    bs = q_ref.shape[0]
    m = mk_ref.shape[0] // bs
    n = negs_ref.shape[0] // bs
    q = q_ref[...]
    mk = mk_ref[...].reshape(bs, m, D)
    mv = mv_ref[...].reshape(bs, m, D)
    negs = negs_ref[...].reshape(bs, n, D)
    resp = resp_ref[...]
    wl = wl_ref[...]

    num = jnp.sum(q[:, None, :] * mk, axis=2)
    qn = jnp.sqrt(jnp.sum(q * q, axis=1))
    mkn = jnp.sqrt(jnp.sum(mk * mk, axis=2))
    den = jnp.maximum(qn, 1e-8)[:, None] * jnp.maximum(mkn, 1e-8)
    sim = num / den
    sm = jax.nn.softmax(sim, axis=1)
    vr = jnp.sum(sm[:, :, None] * mv, axis=1)
    res = jnp.dot(vr, wl.T, preferred_element_type=jnp.float32)
    xe_ref[...] = jnp.broadcast_to(res[:, None, :], (bs, n, D))
    midx = lax.broadcasted_iota(jnp.int32, (bs, n, D), 1)
    ne_ref[...] = jnp.where(midx == 0, resp[:, None, :], negs)


def _dense(pooled_in, pooled_out, w_lin, b, m, n):
    bs = 128
    grid = (b // bs,)
    mk_blocks = b * m // (bs * m)
    q_block0 = 2 * b * m // bs
    resp_block0 = b * n // bs
    return pl.pallas_call(
        _dense_body,
        grid=grid,
        in_specs=[
            pl.BlockSpec((bs * m, D), lambda i: (i, 0)),
            pl.BlockSpec((bs * m, D), lambda i: (i + mk_blocks, 0)),
            pl.BlockSpec((bs, D), lambda i: (i + q_block0, 0)),
            pl.BlockSpec((bs * n, D), lambda i: (i, 0)),
            pl.BlockSpec((bs, D), lambda i: (i + resp_block0, 0)),
            pl.BlockSpec((D, D), lambda i: (0, 0)),
        ],
        out_specs=[
            pl.BlockSpec((bs, n, D), lambda i: (i, 0, 0)),
            pl.BlockSpec((bs, n, D), lambda i: (i, 0, 0)),
        ],
        out_shape=[
            jax.ShapeDtypeStruct((b, n, D), jnp.float32),
            jax.ShapeDtypeStruct((b, n, D), jnp.float32),
        ],
    )(pooled_in, pooled_in, pooled_in, pooled_out, pooled_out, w_lin)


def kernel(query, response, memory_keys, memory_values, negs, W_in, W_out,
           W_lin):
    b, l = query.shape
    m = memory_keys.shape[1]
    n = negs.shape[1]

    v = W_in.shape[0]
    vp = ((v + 12799) // 12800) * 12800
    pad = ((0, vp - v), (0, 0))
    wn_in128, wn_out128 = _renorm2(
        jnp.pad(W_in, pad).reshape(vp // 8, 8 * D),
        jnp.pad(W_out, pad).reshape(vp // 8, 8 * D))
    wn_in = wn_in128.reshape(vp, D // 2)
    wn_out = wn_out128.reshape(vp, D // 2)

    s_in = 2 * b * m + b
    s_out = b * n + b
    pooled_in, pooled_out = _gather_mean(
        wn_in, wn_out,
        memory_keys.reshape(b, m * l), memory_values.reshape(b, m * l),
        query.reshape(-1), negs.reshape(b, n * l), response.reshape(-1),
        s_in, s_out,
    )

    return _dense(pooled_in, pooled_out, W_lin, b, m, n)


# speedup vs baseline: 1.1912x; 1.1912x over previous
import functools

import jax
import jax.numpy as jnp
from jax import lax
from jax.experimental import pallas as pl
from jax.experimental.pallas import tpu as pltpu
from jax.experimental.pallas import tpu_sc as plsc

NW = 32
CH = 32
L = 20
D = 32
IDX_MINOR = 128
CHL = CH * L
NG = CHL // IDX_MINOR


def _renorm_body(a_ref, b_ref, oa_ref, ob_ref):
    gi = lax.broadcasted_iota(jnp.int32, (128, 128), 0) >> 5
    gj = lax.broadcasted_iota(jnp.int32, (128, 128), 1) >> 5
    g = jnp.where(gi == gj, 1.0, 0.0).astype(jnp.float32)

    def renorm(x):
        n2 = jnp.dot(x * x, g, preferred_element_type=jnp.float32)
        n = jnp.sqrt(n2)
        scale = jnp.where(n > 10.0, 10.0 / (n + 1e-7), 1.0)
        return x * scale

    oa_ref[...] = renorm(a_ref[...])
    ob_ref[...] = renorm(b_ref[...])


def _renorm2(w_a, w_b):
    v4 = w_a.shape[0]
    bs = 5000
    spec = pl.BlockSpec((bs, 128), lambda i: (i, 0))
    return pl.pallas_call(
        _renorm_body,
        grid=(v4 // bs,),
        in_specs=[spec, spec],
        out_specs=[spec, spec],
        out_shape=[
            jax.ShapeDtypeStruct((v4, 128), jnp.float32),
            jax.ShapeDtypeStruct((v4, 128), jnp.float32),
        ],
    )(w_a, w_b)


MAXSEG = 50


def _windows(total):
    out = []
    off = 0
    while total - off > 128:
        out.append((off, 128))
        off += 128
    out.append((off, total - off))
    return out


def _gather_mean(table_in, table_out, mk_idx, mv_idx, q_idx, negs_idx,
                 resp_idx, s_in, s_out):
    mesh = plsc.VectorSubcoreMesh(core_axis_name="c", subcore_axis_name="s")

    @functools.partial(
        pl.kernel,
        mesh=mesh,
        compiler_params=pltpu.CompilerParams(use_tc_tiling_on_sc=False),
        out_type=[
            jax.ShapeDtypeStruct((s_in, D), jnp.float32),
            jax.ShapeDtypeStruct((s_out, D), jnp.float32),
        ],
        scratch_types=[
            pltpu.VMEM((MAXSEG * L,), jnp.int32),
            pltpu.VMEM((MAXSEG * L,), jnp.int32),
            pltpu.VMEM((MAXSEG * L,), jnp.int32),
            pltpu.VMEM((MAXSEG * L, D), jnp.float32),
            pltpu.VMEM((MAXSEG * L, D), jnp.float32),
            pltpu.VMEM((MAXSEG * L, D), jnp.float32),
            pltpu.VMEM((MAXSEG, D), jnp.float32),
            pltpu.VMEM((20, D), jnp.float32),
            pltpu.VMEM((32, D), jnp.float32),
            pltpu.SemaphoreType.DMA,
            pltpu.SemaphoreType.DMA,
            pltpu.SemaphoreType.DMA,
        ],
    )
    def k(tin, tout, mki, mvi, qi, ngi, rsi, pooled_in, pooled_out,
          idx0, idx1, idx2, rows0, rows1, rows2, out50, out20, out32,
          sem0, sem1, sem2):
        wid = lax.axis_index("s") * 2 + lax.axis_index("c")
        bufs = ((idx0, rows0, sem0), (idx1, rows1, sem1),
                (idx2, rows2, sem2))

        def gather_descs(table, p, nidx):
            idxv, rowsv, sem = bufs[p]
            return [
                (
                    table.at[idxv.at[pl.ds(off, sz)]],
                    rowsv.at[pl.ds(off, sz), :],
                    sem,
                )
                for off, sz in _windows(nidx)
            ]

        def prefetch(table, idx_row, p, nidx):
            pltpu.sync_copy(idx_row, bufs[p][0].at[pl.ds(0, nidx)])
            for src, dst, sem in gather_descs(table, p, nidx):
                pltpu.async_copy(src, dst, sem)

        def reduce_store(rowsv, outv, segs):
            def halves(r):
                return rowsv[r, pl.ds(0, 16)], rowsv[r, pl.ds(16, 16)]

            def seg_body(s, carry):
                base = s * L
                acc0, acc1 = halves(base)
                for l in range(1, L):
                    a, bb = halves(base + l)
                    acc0 = acc0 + a
                    acc1 = acc1 + bb
                outv[s, pl.ds(0, 16)] = acc0 * (1.0 / L)
                outv[s, pl.ds(16, 16)] = acc1 * (1.0 / L)
                return carry

            lax.fori_loop(0, segs, seg_body, 0)

        def process(table, pooled, outv, row0, p, segs):
            for src, dst, sem in gather_descs(table, p, segs * L):
                pltpu.make_async_copy(src, dst, sem).wait()
            reduce_store(bufs[p][1], outv, segs)
            pltpu.sync_copy(outv, pooled.at[pl.ds(row0, segs), :])

        def do_batch_stream(table, idx2d, pooled, outv, row0, segs):
            b0 = wid * 32

            def pre(c, p):
                prefetch(table, idx2d.at[b0 + c], p, segs * L)

            def proc(c, p):
                process(table, pooled, outv, row0 + (b0 + c) * segs, p,
                        segs)

            pre(0, 0)
            pre(1, 1)
            pre(2, 2)

            def ring_body(k3, carry):
                c = 3 * k3
                proc(c, 0)
                pre(c + 3, 0)
                proc(c + 1, 1)
                pre(c + 4, 1)
                proc(c + 2, 2)
                pre(c + 5, 2)
                return carry

            lax.fori_loop(0, 9, ring_body, 0)
            proc(27, 0)
            pre(30, 0)
            proc(28, 1)
            pre(31, 1)
            proc(29, 2)
            proc(30, 0)
            proc(31, 1)

        def do_flat_stream(table, idx_flat, pooled, row0):
            segs = 32
            b0 = wid * segs
            prefetch(table, idx_flat.at[pl.ds(b0 * L, segs * L)], 0,
                     segs * L)
            process(table, pooled, out32, row0 + b0, 0, segs)

        b = mki.shape[0]
        m = mki.shape[1] // L
        n = ngi.shape[1] // L
        do_batch_stream(tin, mki, pooled_in, out50, 0, m)
        do_batch_stream(tin, mvi, pooled_in, out50, b * m, m)
        do_flat_stream(tin, qi, pooled_in, 2 * b * m)
        do_batch_stream(tout, ngi, pooled_out, out20, 0, n)
        do_flat_stream(tout, rsi, pooled_out, b * n)

    return k(table_in, table_out, mk_idx, mv_idx, q_idx, negs_idx, resp_idx)


def _dense_body(mk_ref, mv_ref, q_ref, negs_ref, resp_ref, wl_ref,
                xe_ref, ne_ref):
    bs = q_ref.shape[0]
    m = mk_ref.shape[0] // bs
    n = negs_ref.shape[0] // bs
    q = q_ref[...]
    mk = mk_ref[...].reshape(bs, m, D)
    mv = mv_ref[...].reshape(bs, m, D)
    negs = negs_ref[...].reshape(bs, n, D)
    resp = resp_ref[...]
    wl = wl_ref[...]

    num = jnp.sum(q[:, None, :] * mk, axis=2)
    qn = jnp.sqrt(jnp.sum(q * q, axis=1))
    mkn = jnp.sqrt(jnp.sum(mk * mk, axis=2))
    den = jnp.maximum(qn, 1e-8)[:, None] * jnp.maximum(mkn, 1e-8)
    sim = num / den
    sm = jax.nn.softmax(sim, axis=1)
    vr = jnp.sum(sm[:, :, None] * mv, axis=1)
    res = jnp.dot(vr, wl.T, preferred_element_type=jnp.float32)
    xe_ref[...] = jnp.broadcast_to(res[:, None, :], (bs, n, D))
    midx = lax.broadcasted_iota(jnp.int32, (bs, n, D), 1)
    ne_ref[...] = jnp.where(midx == 0, resp[:, None, :], negs)


def _dense(pooled_in, pooled_out, w_lin, b, m, n):
    bs = 128
    grid = (b // bs,)
    mk_blocks = b * m // (bs * m)
    q_block0 = 2 * b * m // bs
    resp_block0 = b * n // bs
    return pl.pallas_call(
        _dense_body,
        grid=grid,
        in_specs=[
            pl.BlockSpec((bs * m, D), lambda i: (i, 0)),
            pl.BlockSpec((bs * m, D), lambda i: (i + mk_blocks, 0)),
            pl.BlockSpec((bs, D), lambda i: (i + q_block0, 0)),
            pl.BlockSpec((bs * n, D), lambda i: (i, 0)),
            pl.BlockSpec((bs, D), lambda i: (i + resp_block0, 0)),
            pl.BlockSpec((D, D), lambda i: (0, 0)),
        ],
        out_specs=[
            pl.BlockSpec((bs, n, D), lambda i: (i, 0, 0)),
            pl.BlockSpec((bs, n, D), lambda i: (i, 0, 0)),
        ],
        out_shape=[
            jax.ShapeDtypeStruct((b, n, D), jnp.float32),
            jax.ShapeDtypeStruct((b, n, D), jnp.float32),
        ],
    )(pooled_in, pooled_in, pooled_in, pooled_out, pooled_out, w_lin)


def kernel(query, response, memory_keys, memory_values, negs, W_in, W_out,
           W_lin):
    b, l = query.shape
    m = memory_keys.shape[1]
    n = negs.shape[1]

    v = W_in.shape[0]
    wn_in128, wn_out128 = _renorm2(W_in.reshape(v // 4, 4 * D),
                                   W_out.reshape(v // 4, 4 * D))
    wn_in = wn_in128.reshape(v, D)
    wn_out = wn_out128.reshape(v, D)

    s_in = 2 * b * m + b
    s_out = b * n + b
    pooled_in, pooled_out = _gather_mean(
        wn_in, wn_out,
        memory_keys.reshape(b, m * l), memory_values.reshape(b, m * l),
        query.reshape(-1), negs.reshape(b, n * l), response.reshape(-1),
        s_in, s_out,
    )

    return _dense(pooled_in, pooled_out, W_lin, b, m, n)


# speedup vs baseline: 1.3049x; 1.0955x over previous
import functools

import jax
import jax.numpy as jnp
from jax import lax
from jax.experimental import pallas as pl
from jax.experimental.pallas import tpu as pltpu
from jax.experimental.pallas import tpu_sc as plsc

NW = 32
CH = 32
L = 20
D = 32
IDX_MINOR = 128
CHL = CH * L
NG = CHL // IDX_MINOR


def _renorm_body(a_ref, b_ref, oa_ref, ob_ref):
    gi = lax.broadcasted_iota(jnp.int32, (128, 128), 0) >> 5
    gj = lax.broadcasted_iota(jnp.int32, (128, 128), 1) >> 5
    g = jnp.where(gi == gj, 1.0, 0.0).astype(jnp.float32)

    def renorm(x):
        n2 = jnp.dot(x * x, g, preferred_element_type=jnp.float32)
        n = jnp.sqrt(n2)
        scale = jnp.where(n > 10.0, 10.0 / (n + 1e-7), 1.0)
        return x * scale

    oa_ref[...] = renorm(a_ref[...])
    ob_ref[...] = renorm(b_ref[...])


def _renorm2(w_a, w_b):
    v4 = w_a.shape[0]
    bs = 5000
    spec = pl.BlockSpec((bs, 128), lambda i: (i, 0))
    return pl.pallas_call(
        _renorm_body,
        grid=(v4 // bs,),
        in_specs=[spec, spec],
        out_specs=[spec, spec],
        out_shape=[
            jax.ShapeDtypeStruct((v4, 128), jnp.float32),
            jax.ShapeDtypeStruct((v4, 128), jnp.float32),
        ],
    )(w_a, w_b)


MAXSEG = 50


def _windows(total):
    out = []
    off = 0
    while total - off > 128:
        out.append((off, 128))
        off += 128
    out.append((off, total - off))
    return out


def _gather_mean(table_in, table_out, mk_idx, mv_idx, q_idx, negs_idx,
                 resp_idx, s_in, s_out):
    mesh = plsc.VectorSubcoreMesh(core_axis_name="c", subcore_axis_name="s")

    @functools.partial(
        pl.kernel,
        mesh=mesh,
        compiler_params=pltpu.CompilerParams(use_tc_tiling_on_sc=False),
        out_type=[
            jax.ShapeDtypeStruct((s_in, 128), jnp.float32),
            jax.ShapeDtypeStruct((s_out, 128), jnp.float32),
        ],
        scratch_types=[
            pltpu.VMEM((MAXSEG * L,), jnp.int32),
            pltpu.VMEM((MAXSEG * L,), jnp.int32),
            pltpu.VMEM((MAXSEG * L, D), jnp.float32),
            pltpu.VMEM((MAXSEG * L, D), jnp.float32),
            pltpu.VMEM((13, 128), jnp.float32),
            pltpu.VMEM((5, 128), jnp.float32),
            pltpu.VMEM((32, 128), jnp.float32),
            pltpu.SemaphoreType.DMA,
            pltpu.SemaphoreType.DMA,
        ],
    )
    def k(tin, tout, mki, mvi, qi, ngi, rsi, pooled_in, pooled_out,
          idx0, idx1, rows0, rows1, out13, out5, out32, sem0, sem1):
        wid = lax.axis_index("s") * 2 + lax.axis_index("c")
        bufs = ((idx0, rows0, sem0), (idx1, rows1, sem1))

        def gather_descs(table, p, nidx):
            idxv, rowsv, sem = bufs[p]
            return [
                (
                    table.at[idxv.at[pl.ds(off, sz)]],
                    rowsv.at[pl.ds(off, sz), :],
                    sem,
                )
                for off, sz in _windows(nidx)
            ]

        def prefetch(table, idx_row, p, nidx):
            pltpu.sync_copy(idx_row, bufs[p][0].at[pl.ds(0, nidx)])
            for src, dst, sem in gather_descs(table, p, nidx):
                pltpu.async_copy(src, dst, sem)

        def seg_mean(rowsv, s):
            base = s * L
            acc0 = rowsv[base, pl.ds(0, 16)]
            acc1 = rowsv[base, pl.ds(16, 16)]
            for l in range(1, L):
                acc0 = acc0 + rowsv[base + l, pl.ds(0, 16)]
                acc1 = acc1 + rowsv[base + l, pl.ds(16, 16)]
            return acc0 * (1.0 / L), acc1 * (1.0 / L)

        def reduce_packed(rowsv, outv, segs):
            def grp_body(g2, carry):
                for t in range(4):
                    acc0, acc1 = seg_mean(rowsv, g2 * 4 + t)
                    outv[g2, pl.ds(32 * t, 16)] = acc0
                    outv[g2, pl.ds(32 * t + 16, 16)] = acc1
                return carry

            lax.fori_loop(0, segs // 4, grp_body, 0)
            rem = segs % 4
            if rem:
                row = segs // 4
                for t in range(rem):
                    acc0, acc1 = seg_mean(rowsv, (segs // 4) * 4 + t)
                    outv[row, pl.ds(32 * t, 16)] = acc0
                    outv[row, pl.ds(32 * t + 16, 16)] = acc1
                zero = jnp.zeros((16,), jnp.float32)
                for lane in range(32 * rem, 128, 16):
                    outv[row, pl.ds(lane, 16)] = zero

        def reduce_tiled(rowsv, outv, segs):
            def seg_body(s, carry):
                acc0, acc1 = seg_mean(rowsv, s)
                for t in range(4):
                    outv[s, pl.ds(32 * t, 16)] = acc0
                    outv[s, pl.ds(32 * t + 16, 16)] = acc1
                return carry

            lax.fori_loop(0, segs, seg_body, 0)

        def process(table, pooled, outv, row0, p, segs, reducer):
            for src, dst, sem in gather_descs(table, p, segs * L):
                pltpu.make_async_copy(src, dst, sem).wait()
            reducer(bufs[p][1], outv, segs)
            pltpu.sync_copy(outv, pooled.at[pl.ds(row0, outv.shape[0]), :])

        def do_batch_stream(table, idx2d, pooled, outv, row00, segs):
            b0 = wid * 32
            nrow = outv.shape[0]

            def pre(c, p):
                prefetch(table, idx2d.at[b0 + c], p, segs * L)

            def proc(c, p):
                process(table, pooled, outv, row00 + (b0 + c) * nrow, p,
                        segs, reduce_packed)

            pre(0, 0)
            pre(1, 1)

            def pair_body(k2, carry):
                c = 2 * k2

                proc(c, 0)

                @pl.when(c + 2 < 32)
                def _():
                    pre(c + 2, 0)

                proc(c + 1, 1)

                @pl.when(c + 3 < 32)
                def _():
                    pre(c + 3, 1)

                return carry

            lax.fori_loop(0, 16, pair_body, 0)

        def do_flat_stream(table, idx_flat, pooled, row0):
            segs = 32
            b0 = wid * segs
            prefetch(table, idx_flat.at[pl.ds(b0 * L, segs * L)], 0,
                     segs * L)
            process(table, pooled, out32, row0 + b0, 0, segs, reduce_tiled)

        b = mki.shape[0]
        m = mki.shape[1] // L
        n = ngi.shape[1] // L
        mrow = (m + 3) // 4
        nrow = n // 4
        do_batch_stream(tin, mki, pooled_in, out13, 0, m)
        do_batch_stream(tin, mvi, pooled_in, out13, b * mrow, m)
        do_flat_stream(tin, qi, pooled_in, 2 * b * mrow)
        do_batch_stream(tout, ngi, pooled_out, out5, 0, n)
        do_flat_stream(tout, rsi, pooled_out, b * nrow)

    return k(table_in, table_out, mk_idx, mv_idx, q_idx, negs_idx, resp_idx)


def _dense_body(mk_ref, mv_ref, q4_ref, negs_ref, resp4_ref, wl_ref,
                xe_ref, ne_ref):
    bs = q4_ref.shape[0]
    mrow = mk_ref.shape[0] // bs
    nrow = negs_ref.shape[0] // bs
    m = 50
    gi = lax.broadcasted_iota(jnp.int32, (128, 128), 0) >> 5
    gj = lax.broadcasted_iota(jnp.int32, (128, 128), 1) >> 5
    g = jnp.where(gi == gj, 1.0, 0.0).astype(jnp.float32)

    def group_sums(x2d):
        return jnp.dot(x2d, g, preferred_element_type=jnp.float32)

    mk = mk_ref[...]
    mv3 = mv_ref[...].reshape(bs, mrow, 128)
    q4 = q4_ref[...]

    qn2 = group_sums(q4 * q4)
    q_big = jnp.broadcast_to(q4.reshape(bs, 1, 128), (bs, mrow, 128))
    q_flat = q_big.reshape(bs * mrow, 128)
    num = group_sums(mk * q_flat).reshape(bs, mrow, 128)
    mkn2 = group_sums(mk * mk).reshape(bs, mrow, 128)

    qn = jnp.maximum(jnp.sqrt(qn2), 1e-8)
    den = qn.reshape(bs, 1, 128) * jnp.maximum(jnp.sqrt(mkn2), 1e-8)
    sim = num / den

    jrow = lax.broadcasted_iota(jnp.int32, (bs, mrow, 128), 1)
    lane = lax.broadcasted_iota(jnp.int32, (bs, mrow, 128), 2)
    m_id = jrow * 4 + (lane >> 5)
    simm = jnp.where(m_id < m, sim, -1e30)
    mx = jnp.max(jnp.max(simm, axis=2), axis=1)
    e = jnp.exp(simm - mx.reshape(bs, 1, 1))
    z = jnp.sum(jnp.sum(e, axis=2), axis=1) * (1.0 / 32.0)
    sm = e / z.reshape(bs, 1, 1)

    s128 = jnp.sum(sm * mv3, axis=1)
    vr = (s128[:, 0:32] + s128[:, 32:64] + s128[:, 64:96]
          + s128[:, 96:128])
    wl = wl_ref[...]
    res = jnp.dot(vr, wl.T, preferred_element_type=jnp.float32)

    n = 4 * nrow
    xe_ref[...] = jnp.broadcast_to(res.reshape(bs, 1, D), (bs, n, D))
    negs3 = negs_ref[...].reshape(bs, nrow, 128)
    resp = resp4_ref[...][:, 0:32]
    ne_ref[:, 0, :] = resp
    for mm in range(1, n):
        ne_ref[:, mm, :] = negs3[:, mm // 4, 32 * (mm % 4):32 * (mm % 4) + 32]


def _dense(pooled_in, pooled_out, w_lin, b, m, n):
    bs = 128
    grid = (b // bs,)
    mrow = (m + 3) // 4
    nrow = n // 4
    mk_blocks = b * mrow // (bs * mrow)
    q_block0 = 2 * b * mrow // bs
    resp_block0 = b * nrow // bs
    return pl.pallas_call(
        _dense_body,
        grid=grid,
        in_specs=[
            pl.BlockSpec((bs * mrow, 128), lambda i: (i, 0)),
            pl.BlockSpec((bs * mrow, 128), lambda i: (i + mk_blocks, 0)),
            pl.BlockSpec((bs, 128), lambda i: (i + q_block0, 0)),
            pl.BlockSpec((bs * nrow, 128), lambda i: (i, 0)),
            pl.BlockSpec((bs, 128), lambda i: (i + resp_block0, 0)),
            pl.BlockSpec((D, D), lambda i: (0, 0)),
        ],
        out_specs=[
            pl.BlockSpec((bs, n, D), lambda i: (i, 0, 0)),
            pl.BlockSpec((bs, n, D), lambda i: (i, 0, 0)),
        ],
        out_shape=[
            jax.ShapeDtypeStruct((b, n, D), jnp.float32),
            jax.ShapeDtypeStruct((b, n, D), jnp.float32),
        ],
    )(pooled_in, pooled_in, pooled_in, pooled_out, pooled_out, w_lin)


def kernel(query, response, memory_keys, memory_values, negs, W_in, W_out,
           W_lin):
    b, l = query.shape
    m = memory_keys.shape[1]
    n = negs.shape[1]

    v = W_in.shape[0]
    wn_in128, wn_out128 = _renorm2(W_in.reshape(v // 4, 4 * D),
                                   W_out.reshape(v // 4, 4 * D))
    wn_in = wn_in128.reshape(v, D)
    wn_out = wn_out128.reshape(v, D)

    s_in = 2 * b * ((m + 3) // 4) + b
    s_out = b * (n // 4) + b
    pooled_in, pooled_out = _gather_mean(
        wn_in, wn_out,
        memory_keys.reshape(b, m * l), memory_values.reshape(b, m * l),
        query.reshape(-1), negs.reshape(b, n * l), response.reshape(-1),
        s_in, s_out,
    )

    return _dense(pooled_in, pooled_out, W_lin, b, m, n)
